# baseline (speedup 1.0000x reference)
import functools

import jax
import jax.numpy as jnp
from jax import lax
from jax.experimental import pallas as pl
from jax.experimental.pallas import tpu as pltpu
from jax.experimental.pallas import tpu_sc as plsc

N = 10000
E = 320000
D = 128

NC = 2
NS = 16
NW = NC * NS
EPW = E // NW
CHUNK = 80
NCHUNK = EPW // CHUNK
NBUF = 3
NFULL = (NCHUNK // NBUF) * NBUF
RPT = 632
RPT_LAST = N - (NS - 1) * RPT

_mesh = plsc.VectorSubcoreMesh(core_axis_name="c", subcore_axis_name="s")


@functools.partial(
    pl.kernel,
    mesh=_mesh,
    out_type=jax.ShapeDtypeStruct((NC, N, D), jnp.float32),
    scratch_types=[
        pltpu.VMEM((NCHUNK, CHUNK), jnp.int32),
        pltpu.VMEM((NCHUNK, CHUNK), jnp.int32),
        pltpu.VMEM((NBUF, CHUNK, D), jnp.float32),
        pltpu.VMEM_SHARED((N, D), jnp.float32),
        pltpu.SemaphoreType.DMA((NBUF,)),
        pltpu.SemaphoreType.DMA((NBUF,)),
    ],
    compiler_params=pltpu.CompilerParams(use_tc_tiling_on_sc=False),
)
def _edge_agg(x_hbm, zeros_hbm, ei_hbm, out_hbm, src_v, dst_v, rows_v, acc,
              gsem, ssem):
    c = lax.axis_index("c")
    s = lax.axis_index("s")
    wid = s * NC + c
    row0 = pl.multiple_of(s * RPT, 8)

    pltpu.sync_copy(ei_hbm.at[0, wid], src_v)
    pltpu.sync_copy(ei_hbm.at[1, wid], dst_v)
    for b in range(NBUF):
        pltpu.async_copy(x_hbm.at[src_v.at[b]], rows_v.at[b], gsem.at[b])

    seed = [x_hbm, zeros_hbm]
    for cc in range(NC):
        @pl.when((c == cc) & (s < NS - 1))
        def _(cc=cc):
            pltpu.sync_copy(seed[cc].at[pl.ds(row0, RPT)],
                            acc.at[pl.ds(row0, RPT)])

        @pl.when((c == cc) & (s == NS - 1))
        def _(cc=cc):
            pltpu.sync_copy(seed[cc].at[pl.ds((NS - 1) * RPT, RPT_LAST)],
                            acc.at[pl.ds((NS - 1) * RPT, RPT_LAST)])

    plsc.subcore_barrier()

    def slot(i, b):
        pltpu.make_async_copy(x_hbm.at[src_v.at[i]], rows_v.at[b],
                              gsem.at[b]).wait()
        pltpu.sync_copy(rows_v.at[b], acc.at[dst_v.at[i]], add=True)

        @pl.when(i + NBUF < NCHUNK)
        def _():
            pltpu.async_copy(x_hbm.at[src_v.at[i + NBUF]], rows_v.at[b],
                             gsem.at[b])

    def step(g, carry):
        for b in range(NBUF):
            slot(g * NBUF + b, b)
        return carry

    lax.fori_loop(0, NFULL // NBUF, step, 0)
    for i in range(NFULL, NCHUNK):
        slot(i, i % NBUF)
    plsc.subcore_barrier()

    @pl.when(s < NS - 1)
    def _():
        pltpu.sync_copy(acc.at[pl.ds(row0, RPT)],
                        out_hbm.at[c, pl.ds(row0, RPT)])

    @pl.when(s == NS - 1)
    def _():
        pltpu.sync_copy(acc.at[pl.ds((NS - 1) * RPT, RPT_LAST)],
                        out_hbm.at[c, pl.ds((NS - 1) * RPT, RPT_LAST)])


def _bn_relu(h, g, b):
    mu = jnp.mean(h, axis=0, keepdims=True)
    d = h - mu
    var = jnp.mean(d * d, axis=0, keepdims=True)
    return jnp.maximum(d * (g * lax.rsqrt(var + 1e-5)) + b, 0.0)


def _dot(a, b):
    return lax.dot_general(a, b, (((1,), (0,)), ((), ())),
                           preferred_element_type=jnp.float32)


def _mlp0_body(p_ref, w1_ref, b1_ref, g1_ref, be1_ref,
               w2_ref, b2_ref, g2_ref, be2_ref, out_ref):
    z = p_ref[0] + p_ref[1]
    h = _dot(z, w1_ref[...]) + b1_ref[...]
    h = _bn_relu(h, g1_ref[...], be1_ref[...])
    h = _dot(h, w2_ref[...]) + b2_ref[...]
    out_ref[...] = _bn_relu(h, g2_ref[...], be2_ref[...])


def _mlp1_body(p_ref, w1_ref, b1_ref, g1_ref, be1_ref,
               w2_ref, b2_ref, out_ref):
    z = p_ref[0] + p_ref[1]
    h = _dot(z, w1_ref[...]) + b1_ref[...]
    h = _bn_relu(h, g1_ref[...], be1_ref[...])
    out_ref[...] = _dot(h, w2_ref[...]) + b2_ref[...]


def kernel(x, edge_index, W1_0, b1_0, g1_0, be1_0, W2_0, b2_0, g2_0, be2_0,
           W1_1, b1_1, g1_1, be1_1, W2_1, b2_1):
    ei = edge_index.reshape(2, NW, NCHUNK, CHUNK)
    zeros = jnp.zeros((N, D), jnp.float32)

    p = _edge_agg(x, zeros, ei)
    h = pl.pallas_call(
        _mlp0_body,
        out_shape=jax.ShapeDtypeStruct((N, D), jnp.float32),
    )(p, W1_0, b1_0.reshape(1, -1), g1_0.reshape(1, -1),
      be1_0.reshape(1, -1), W2_0, b2_0.reshape(1, -1), g2_0.reshape(1, -1),
      be2_0.reshape(1, -1))

    p = _edge_agg(h, zeros, ei)
    out = pl.pallas_call(
        _mlp1_body,
        out_shape=jax.ShapeDtypeStruct((N, D), jnp.float32),
    )(p, W1_1, b1_1.reshape(1, -1), g1_1.reshape(1, -1),
      be1_1.reshape(1, -1), W2_1, b2_1.reshape(1, -1))
    return out


# speedup vs baseline: 1.0256x; 1.0256x over previous
import functools

import jax
import jax.numpy as jnp
from jax import lax
from jax.experimental import pallas as pl
from jax.experimental.pallas import tpu as pltpu
from jax.experimental.pallas import tpu_sc as plsc

N = 10000
E = 320000
D = 128

NC = 2
NS = 16
NW = NC * NS
EPW = E // NW
CHUNK = 40
NCHUNK = EPW // CHUNK
NBUF = 5
NFULL = (NCHUNK // NBUF) * NBUF
RPT = 632
RPT_LAST = N - (NS - 1) * RPT

_mesh = plsc.VectorSubcoreMesh(core_axis_name="c", subcore_axis_name="s")


@functools.partial(
    pl.kernel,
    mesh=_mesh,
    out_type=jax.ShapeDtypeStruct((NC, N, D), jnp.float32),
    scratch_types=[
        pltpu.VMEM((NCHUNK, CHUNK), jnp.int32),
        pltpu.VMEM((NCHUNK, CHUNK), jnp.int32),
        pltpu.VMEM((NBUF, CHUNK, D), jnp.float32),
        pltpu.VMEM_SHARED((N, D), jnp.float32),
        pltpu.SemaphoreType.DMA((NBUF,)),
        pltpu.SemaphoreType.DMA((NBUF,)),
    ],
    compiler_params=pltpu.CompilerParams(use_tc_tiling_on_sc=False),
)
def _edge_agg(x_hbm, zeros_hbm, ei_hbm, out_hbm, src_v, dst_v, rows_v, acc,
              gsem, ssem):
    c = lax.axis_index("c")
    s = lax.axis_index("s")
    wid = s * NC + c
    row0 = pl.multiple_of(s * RPT, 8)

    pltpu.sync_copy(ei_hbm.at[0, wid], src_v)
    pltpu.sync_copy(ei_hbm.at[1, wid], dst_v)
    for b in range(NBUF):
        pltpu.async_copy(x_hbm.at[src_v.at[b]], rows_v.at[b], gsem.at[b])

    seed = [x_hbm, zeros_hbm]
    for cc in range(NC):
        @pl.when((c == cc) & (s < NS - 1))
        def _(cc=cc):
            pltpu.sync_copy(seed[cc].at[pl.ds(row0, RPT)],
                            acc.at[pl.ds(row0, RPT)])

        @pl.when((c == cc) & (s == NS - 1))
        def _(cc=cc):
            pltpu.sync_copy(seed[cc].at[pl.ds((NS - 1) * RPT, RPT_LAST)],
                            acc.at[pl.ds((NS - 1) * RPT, RPT_LAST)])

    plsc.subcore_barrier()

    def slot(i, b):
        pltpu.make_async_copy(x_hbm.at[src_v.at[i]], rows_v.at[b],
                              gsem.at[b]).wait()
        pltpu.sync_copy(rows_v.at[b], acc.at[dst_v.at[i]], add=True)

        @pl.when(i + NBUF < NCHUNK)
        def _():
            pltpu.async_copy(x_hbm.at[src_v.at[i + NBUF]], rows_v.at[b],
                             gsem.at[b])

    def step(g, carry):
        for b in range(NBUF):
            slot(g * NBUF + b, b)
        return carry

    lax.fori_loop(0, NFULL // NBUF, step, 0)
    for i in range(NFULL, NCHUNK):
        slot(i, i % NBUF)
    plsc.subcore_barrier()

    @pl.when(s < NS - 1)
    def _():
        pltpu.sync_copy(acc.at[pl.ds(row0, RPT)],
                        out_hbm.at[c, pl.ds(row0, RPT)])

    @pl.when(s == NS - 1)
    def _():
        pltpu.sync_copy(acc.at[pl.ds((NS - 1) * RPT, RPT_LAST)],
                        out_hbm.at[c, pl.ds((NS - 1) * RPT, RPT_LAST)])


def _bn_relu(h, g, b):
    mu = jnp.mean(h, axis=0, keepdims=True)
    d = h - mu
    var = jnp.mean(d * d, axis=0, keepdims=True)
    return jnp.maximum(d * (g * lax.rsqrt(var + 1e-5)) + b, 0.0)


def _dot(a, b):
    return lax.dot_general(a, b, (((1,), (0,)), ((), ())),
                           preferred_element_type=jnp.float32)


def _mlp0_body(p_ref, w1_ref, b1_ref, g1_ref, be1_ref,
               w2_ref, b2_ref, g2_ref, be2_ref, out_ref):
    z = p_ref[0] + p_ref[1]
    h = _dot(z, w1_ref[...]) + b1_ref[...]
    h = _bn_relu(h, g1_ref[...], be1_ref[...])
    h = _dot(h, w2_ref[...]) + b2_ref[...]
    out_ref[...] = _bn_relu(h, g2_ref[...], be2_ref[...])


def _mlp1_body(p_ref, w1_ref, b1_ref, g1_ref, be1_ref,
               w2_ref, b2_ref, out_ref):
    z = p_ref[0] + p_ref[1]
    h = _dot(z, w1_ref[...]) + b1_ref[...]
    h = _bn_relu(h, g1_ref[...], be1_ref[...])
    out_ref[...] = _dot(h, w2_ref[...]) + b2_ref[...]


def kernel(x, edge_index, W1_0, b1_0, g1_0, be1_0, W2_0, b2_0, g2_0, be2_0,
           W1_1, b1_1, g1_1, be1_1, W2_1, b2_1):
    ei = edge_index.reshape(2, NW, NCHUNK, CHUNK)
    zeros = jnp.zeros((N, D), jnp.float32)

    p = _edge_agg(x, zeros, ei)
    h = pl.pallas_call(
        _mlp0_body,
        out_shape=jax.ShapeDtypeStruct((N, D), jnp.float32),
    )(p, W1_0, b1_0.reshape(1, -1), g1_0.reshape(1, -1),
      be1_0.reshape(1, -1), W2_0, b2_0.reshape(1, -1), g2_0.reshape(1, -1),
      be2_0.reshape(1, -1))

    p = _edge_agg(h, zeros, ei)
    out = pl.pallas_call(
        _mlp1_body,
        out_shape=jax.ShapeDtypeStruct((N, D), jnp.float32),
    )(p, W1_1, b1_1.reshape(1, -1), g1_1.reshape(1, -1),
      be1_1.reshape(1, -1), W2_1, b2_1.reshape(1, -1))
    return out
